# Initial kernel scaffold; baseline (speedup 1.0000x reference)
import jax
import jax.numpy as jnp
from jax.experimental import pallas as pl


def kernel(m, h, u, v, Fxy, Fuv):
    raise NotImplementedError("write your pallas kernel here")


# speedup vs baseline: 1.1093x; 1.1093x over previous
import functools

import jax
import jax.numpy as jnp
from jax import lax
from jax.experimental import pallas as pl
from jax.experimental.pallas import tpu as pltpu
from jax.experimental.pallas import tpu_sc as plsc

_M, _HX, _HY, _LXY = 8, 512, 512, 8
_U, _V, _LUV = 400, 400, 8
_N = 1048576
_LO = _LXY + _LUV

_NC, _NS, _L = 2, 16, 16
_NW = _NC * _NS
_PW = _N // _NW
_C = 512
_NCH = _PW // _C
_NG = _C // _L
_RB = 4 * _C
_IBLK = 128
_NBLK = _RB // _IBLK


def _sc_body(m_hbm, hx_hbm, hy_hbm, u_hbm, v_hbm, fxy_hbm, fuv_hbm,
             out_hbm,
             m_v, hx_v, hy_v, u_v, v_v, idxxy_v, idxuv_v, wxy_v, wuv_v,
             bufxy_v, bufuv_v, out_v, sem_in, sem_xy, sem_uv, sem_out):
    wid = lax.axis_index("s") * _NC + lax.axis_index("c")
    base_w = wid * _PW
    iota = lax.iota(jnp.int32, _L)

    def corners(find, size):
        find = jnp.where(find >= float(size), jnp.full((_L,), float(size - 1)),
                         find)
        i1 = find.astype(jnp.int32)
        fr = find - i1.astype(jnp.float32)
        i2 = i1 + 1
        i2 = jnp.where(i2 >= size, jnp.zeros((_L,), jnp.int32), i2)
        return i1, i2, fr

    @pl.loop(0, _NCH)
    def chunk(k):
        cb = base_w + k * _C

        cm = pltpu.async_copy(m_hbm.at[pl.ds(cb, _C)], m_v, sem_in)
        cx = pltpu.async_copy(hx_hbm.at[pl.ds(cb, _C)], hx_v, sem_in)
        cy = pltpu.async_copy(hy_hbm.at[pl.ds(cb, _C)], hy_v, sem_in)
        cu = pltpu.async_copy(u_hbm.at[pl.ds(cb, _C)], u_v, sem_in)
        cv = pltpu.async_copy(v_hbm.at[pl.ds(cb, _C)], v_v, sem_in)
        cm.wait(); cx.wait(); cy.wait(); cu.wait(); cv.wait()

        @pl.loop(0, _NG)
        def grp(gi):
            off = gi * _L
            mv = m_v[pl.ds(off, _L)]
            hx = hx_v[pl.ds(off, _L)]
            hy = hy_v[pl.ds(off, _L)]
            uu = u_v[pl.ds(off, _L)]
            vv = v_v[pl.ds(off, _L)]

            i1, i2, ir = corners((hx + 1.0) * (0.5 * _HX), _HX)
            j1, j2, jr = corners((hy + 1.0) * (0.5 * _HY), _HY)
            base = mv * (_HX * _HY)
            a1 = base + i1 * _HY
            a2 = base + i2 * _HY
            idxxy_v[pl.ds(0 * _C + off, _L)] = a1 + j1
            idxxy_v[pl.ds(1 * _C + off, _L)] = a2 + j1
            idxxy_v[pl.ds(2 * _C + off, _L)] = a1 + j2
            idxxy_v[pl.ds(3 * _C + off, _L)] = a2 + j2
            omi = 1.0 - ir
            omj = 1.0 - jr
            wxy_v[pl.ds(0 * _C + off, _L)] = omi * omj
            wxy_v[pl.ds(1 * _C + off, _L)] = ir * omj
            wxy_v[pl.ds(2 * _C + off, _L)] = omi * jr
            wxy_v[pl.ds(3 * _C + off, _L)] = ir * jr

            p1, p2, pr = corners(uu * float(_U), _U)
            q1, q2, qr = corners(vv * float(_V), _V)
            baseu = mv * (_U * _V)
            b1 = baseu + p1 * _V
            b2 = baseu + p2 * _V
            idxuv_v[pl.ds(0 * _C + off, _L)] = b1 + q1
            idxuv_v[pl.ds(1 * _C + off, _L)] = b2 + q1
            idxuv_v[pl.ds(2 * _C + off, _L)] = b1 + q2
            idxuv_v[pl.ds(3 * _C + off, _L)] = b2 + q2
            omp = 1.0 - pr
            omq = 1.0 - qr
            wuv_v[pl.ds(0 * _C + off, _L)] = omp * omq
            wuv_v[pl.ds(1 * _C + off, _L)] = pr * omq
            wuv_v[pl.ds(2 * _C + off, _L)] = omp * qr
            wuv_v[pl.ds(3 * _C + off, _L)] = pr * qr

        @pl.loop(0, _NBLK)
        def fire(b):
            o = b * _IBLK
            pltpu.async_copy(fxy_hbm.at[idxxy_v.at[pl.ds(o, _IBLK)]],
                             bufxy_v.at[pl.ds(o, _IBLK)], sem_xy)
            pltpu.async_copy(fuv_hbm.at[idxuv_v.at[pl.ds(o, _IBLK)]],
                             bufuv_v.at[pl.ds(o, _IBLK)], sem_uv)

        @pl.when(k > 0)
        def _():
            pltpu.make_async_copy(out_v, out_hbm.at[pl.ds(0, _C)],
                                  sem_out).wait()

        pltpu.make_async_copy(fxy_hbm.at[pl.ds(0, _RB)], bufxy_v,
                              sem_xy).wait()
        pltpu.make_async_copy(fuv_hbm.at[pl.ds(0, _RB)], bufuv_v,
                              sem_uv).wait()

        @pl.loop(0, _NG)
        def blend(gi):
            off = gi * _L
            pts = off + iota
            for (buf, wv, cbase) in ((bufxy_v, wxy_v, 0),
                                     (bufuv_v, wuv_v, _LXY)):
                w11 = wv[pl.ds(0 * _C + off, _L)]
                w21 = wv[pl.ds(1 * _C + off, _L)]
                w12 = wv[pl.ds(2 * _C + off, _L)]
                w22 = wv[pl.ds(3 * _C + off, _L)]
                r11 = pts
                r21 = pts + 1 * _C
                r12 = pts + 2 * _C
                r22 = pts + 3 * _C
                for l in range(_LXY):
                    col = jnp.full((_L,), l, jnp.int32)
                    g11 = plsc.load_gather(buf, [r11, col])
                    g21 = plsc.load_gather(buf, [r21, col])
                    g12 = plsc.load_gather(buf, [r12, col])
                    g22 = plsc.load_gather(buf, [r22, col])
                    acc = g11 * w11 + g21 * w21 + g12 * w12 + g22 * w22
                    plsc.store_scatter(
                        out_v, [pts, jnp.full((_L,), cbase + l, jnp.int32)],
                        acc)

        orow0 = pl.multiple_of(cb, _C)
        pltpu.async_copy(out_v, out_hbm.at[pl.ds(orow0, _C)], sem_out)

    pltpu.make_async_copy(out_v, out_hbm.at[pl.ds(0, _C)], sem_out).wait()


_sc_kernel = pl.kernel(
    _sc_body,
    out_type=jax.ShapeDtypeStruct((_N, _LO), jnp.float32),
    mesh=plsc.VectorSubcoreMesh(core_axis_name="c", subcore_axis_name="s"),
    compiler_params=pltpu.CompilerParams(needs_layout_passes=False,
                                         use_tc_tiling_on_sc=False),
    scratch_types=[
        pltpu.VMEM((_C,), jnp.int32),
        pltpu.VMEM((_C,), jnp.float32),
        pltpu.VMEM((_C,), jnp.float32),
        pltpu.VMEM((_C,), jnp.float32),
        pltpu.VMEM((_C,), jnp.float32),
        pltpu.VMEM((_RB,), jnp.int32),
        pltpu.VMEM((_RB,), jnp.int32),
        pltpu.VMEM((_RB,), jnp.float32),
        pltpu.VMEM((_RB,), jnp.float32),
        pltpu.VMEM((_RB, _LXY), jnp.float32),
        pltpu.VMEM((_RB, _LUV), jnp.float32),
        pltpu.VMEM((_C, _LO), jnp.float32),
        pltpu.SemaphoreType.DMA,
        pltpu.SemaphoreType.DMA,
        pltpu.SemaphoreType.DMA,
        pltpu.SemaphoreType.DMA,
    ],
)


@jax.jit
def kernel(m, h, u, v, Fxy, Fuv):
    fxy = Fxy.reshape(_M * _HX * _HY, _LXY)
    fuv = Fuv.reshape(_M * _U * _V, _LUV)
    return _sc_kernel(m, h[:, 0], h[:, 1], u, v, fxy, fuv)
